# baseline (speedup 1.0000x reference)
import functools

import jax
import jax.numpy as jnp
import numpy as np
from jax.experimental import pallas as pl
from jax.experimental.pallas import tpu as pltpu
from jax.experimental.pallas import tpu_sc as plsc

DIM = 192
NH = 6
HD = DIM // NH
WS = (6, 8, 8)
N = WS[0] * WS[1] * WS[2]
NW = 36
E = 8
DH = 384
T = 6 * 48 * 48


def _rel_index_np(ws):
    wd, wh, ww = ws
    coords = np.stack(
        np.meshgrid(np.arange(wd), np.arange(wh), np.arange(ww), indexing="ij"))
    cf = coords.reshape(3, -1)
    rel = cf[:, :, None] - cf[:, None, :]
    rel = rel.transpose(1, 2, 0).astype(np.int64)
    rel[..., 0] += wd - 1
    rel[..., 1] += wh - 1
    rel[..., 2] += ww - 1
    rel[..., 0] *= (2 * wh - 1) * (2 * ww - 1)
    rel[..., 1] *= (2 * ww - 1)
    return rel.sum(-1)


_REL_FLAT = _rel_index_np(WS).reshape(-1)


def _ln(x, w, b):
    m = jnp.mean(x, axis=-1, keepdims=True)
    xc = x - m
    v = jnp.mean(xc * xc, axis=-1, keepdims=True)
    return xc * jax.lax.rsqrt(v + 1e-5) * w + b


def _attn_body(xw_ref, bias_ref, n1w_ref, n1b_ref, qkvwt_ref, qkvb_ref,
               projwt_ref, projb_ref, out_ref):
    x = xw_ref[0]
    xn = _ln(x, n1w_ref[0], n1b_ref[0])
    qkv = jnp.dot(xn, qkvwt_ref[...],
                  preferred_element_type=jnp.float32) + qkvb_ref[0]
    scale = HD ** -0.5
    outs = []
    for h in range(NH):
        q = qkv[:, h * HD:(h + 1) * HD] * scale
        k = qkv[:, DIM + h * HD:DIM + (h + 1) * HD]
        v = qkv[:, 2 * DIM + h * HD:2 * DIM + (h + 1) * HD]
        s = jnp.dot(q, k.T, preferred_element_type=jnp.float32) + bias_ref[h]
        s = s - jnp.max(s, axis=-1, keepdims=True)
        p = jnp.exp(s)
        p = p / jnp.sum(p, axis=-1, keepdims=True)
        outs.append(jnp.dot(p, v, preferred_element_type=jnp.float32))
    o = jnp.concatenate(outs, axis=-1)
    out_ref[0] = x + jnp.dot(o, projwt_ref[...],
                             preferred_element_type=jnp.float32) + projb_ref[0]


def _attention(xw, bias, norm1_w, norm1_b, qkv_w, qkv_b, proj_w, proj_b):
    qkv_wt = qkv_w.T
    proj_wt = proj_w.T
    return pl.pallas_call(
        _attn_body,
        grid=(NW,),
        in_specs=[
            pl.BlockSpec((1, N, DIM), lambda i: (i, 0, 0)),
            pl.BlockSpec((NH, N, N), lambda i: (0, 0, 0)),
            pl.BlockSpec((1, DIM), lambda i: (0, 0)),
            pl.BlockSpec((1, DIM), lambda i: (0, 0)),
            pl.BlockSpec((DIM, 3 * DIM), lambda i: (0, 0)),
            pl.BlockSpec((1, 3 * DIM), lambda i: (0, 0)),
            pl.BlockSpec((DIM, DIM), lambda i: (0, 0)),
            pl.BlockSpec((1, DIM), lambda i: (0, 0)),
        ],
        out_specs=pl.BlockSpec((1, N, DIM), lambda i: (i, 0, 0)),
        out_shape=jax.ShapeDtypeStruct((NW, N, DIM), jnp.float32),
        compiler_params=pltpu.CompilerParams(
            dimension_semantics=("parallel",)),
    )(xw, bias, norm1_w.reshape(1, DIM), norm1_b.reshape(1, DIM), qkv_wt,
      qkv_b.reshape(1, 3 * DIM), proj_wt, proj_b.reshape(1, DIM))


def _gelu(x):
    return 0.5 * x * (1.0 + jax.lax.erf(x * (2.0 ** -0.5)))


def _top2(logits):
    M = logits.shape[0]
    m0 = jnp.full((M, 1), -jnp.inf, jnp.float32)
    e0 = jnp.zeros((M, 1), jnp.int32)
    for e in range(E):
        le = logits[:, e:e + 1]
        c = le > m0
        e0 = jnp.where(c, e, e0)
        m0 = jnp.where(c, le, m0)
    m1 = jnp.full((M, 1), -jnp.inf, jnp.float32)
    e1 = jnp.zeros((M, 1), jnp.int32)
    for e in range(E):
        le = logits[:, e:e + 1]
        c = (le > m1) & (e0 != e)
        e1 = jnp.where(c, e, e1)
        m1 = jnp.where(c, le, m1)
    g0 = 1.0 / (1.0 + jnp.exp(m1 - m0))
    g1 = 1.0 - g0
    return g0, g1, e0, e1


BT = 256
NT = (2 * T + E * (BT - 1) + BT - 1) // BT
L = NT * BT
NC = T // 128
DP = 256


TMA = 1728


def _gate_body(x1_ref, n2w_ref, n2b_ref, gw_ref, tln_ref, es_ref, gwt_ref):
    x = x1_ref[...]
    t = _ln(x, n2w_ref[0], n2b_ref[0])
    tln_ref[...] = jnp.concatenate(
        [t, jnp.zeros((TMA, DP - DIM), jnp.float32)], axis=-1)
    logitsT = jax.lax.dot_general(gw_ref[...], t, (((1,), (1,)), ((), ())),
                                  preferred_element_type=jnp.float32)
    m0 = jnp.full((1, TMA), -jnp.inf, jnp.float32)
    e0 = jnp.zeros((1, TMA), jnp.float32)
    for e in range(E):
        le = logitsT[e:e + 1, :]
        c = le > m0
        e0 = jnp.where(c, float(e), e0)
        m0 = jnp.where(c, le, m0)
    m1 = jnp.full((1, TMA), -jnp.inf, jnp.float32)
    e1 = jnp.zeros((1, TMA), jnp.float32)
    for e in range(E):
        le = logitsT[e:e + 1, :]
        c = (le > m1) & (e0 != float(e))
        e1 = jnp.where(c, float(e), e1)
        m1 = jnp.where(c, le, m1)
    g0 = 1.0 / (1.0 + jnp.exp(m1 - m0))
    es_ref[0] = jnp.concatenate([e0, e1], axis=0)
    gwt_ref[0] = jnp.concatenate([g0, 1.0 - g0], axis=0)


def _gate(x1, norm2_w, norm2_b, gate_w):
    return pl.pallas_call(
        _gate_body,
        grid=(T // TMA,),
        in_specs=[
            pl.BlockSpec((TMA, DIM), lambda i: (i, 0)),
            pl.BlockSpec((1, DIM), lambda i: (0, 0)),
            pl.BlockSpec((1, DIM), lambda i: (0, 0)),
            pl.BlockSpec((E, DIM), lambda i: (0, 0)),
        ],
        out_specs=[
            pl.BlockSpec((TMA, DP), lambda i: (i, 0)),
            pl.BlockSpec((1, 2, TMA), lambda i: (i, 0, 0)),
            pl.BlockSpec((1, 2, TMA), lambda i: (i, 0, 0)),
        ],
        out_shape=[
            jax.ShapeDtypeStruct((T, DP), jnp.float32),
            jax.ShapeDtypeStruct((T // TMA, 2, TMA), jnp.float32),
            jax.ShapeDtypeStruct((T // TMA, 2, TMA), jnp.float32),
        ],
        compiler_params=pltpu.CompilerParams(
            dimension_semantics=("parallel",)),
    )(x1, norm2_w.reshape(1, DIM), norm2_b.reshape(1, DIM), gate_w)


def _pos_body(es_ref, pos_ref, tm_ref):
    e0 = es_ref[0:1, :]
    e1 = es_ref[1:2, :]
    eio = jax.lax.broadcasted_iota(jnp.int32, (E, 1), 0).astype(jnp.float32)
    oh0 = (eio == e0).astype(jnp.float32)
    oh1 = (eio == e1).astype(jnp.float32)

    i0 = jax.lax.broadcasted_iota(jnp.int32, (128, 128), 0)
    i1 = jax.lax.broadcasted_iota(jnp.int32, (128, 128), 1)
    teu128 = (i0 < i1).astype(jnp.float32)
    j0 = jax.lax.broadcasted_iota(jnp.int32, (NC, NC), 0)
    j1 = jax.lax.broadcasted_iota(jnp.int32, (NC, NC), 1)
    teuN = (j0 < j1).astype(jnp.float32)

    def ranks(oh):
        ohc = oh.reshape(E, NC, 128)
        intra = jax.lax.dot_general(ohc, teu128, (((2,), (0,)), ((), ())),
                                    preferred_element_type=jnp.float32)
        s = jnp.sum(ohc, axis=2)
        cs = jax.lax.dot_general(s, teuN, (((1,), (0,)), ((), ())),
                                 preferred_element_type=jnp.float32)
        r = intra + cs[:, :, None]
        return r.reshape(E, T), jnp.sum(s, axis=1, keepdims=True)

    r0, c0 = ranks(oh0)
    r1, c1 = ranks(oh1)
    counts = c0 + c1
    padded = jnp.floor((counts + (BT - 1)) * (1.0 / BT)) * BT
    k0 = jax.lax.broadcasted_iota(jnp.int32, (E, E), 0)
    k1 = jax.lax.broadcasted_iota(jnp.int32, (E, E), 1)
    tel8 = (k1 < k0).astype(jnp.float32)
    pstart = jax.lax.dot_general(tel8, padded, (((1,), (0,)), ((), ())),
                                 preferred_element_type=jnp.float32)
    pend = pstart + padded

    def sel(oh, v):
        return jnp.sum(oh * v, axis=0, keepdims=True)

    pos0 = sel(oh0, pstart) + jnp.sum(oh0 * r0, axis=0, keepdims=True)
    pos1 = (sel(oh1, pstart) + sel(oh1, c0)
            + jnp.sum(oh1 * r1, axis=0, keepdims=True))
    pos_ref[...] = jnp.concatenate([pos0, pos1], axis=0).astype(jnp.int32)
    ti = jax.lax.broadcasted_iota(jnp.int32, (1, NT), 1).astype(jnp.float32) * BT
    tm = jnp.sum((pend <= ti).astype(jnp.int32), axis=0, keepdims=True)
    tm_ref[...] = jnp.minimum(tm, E - 1)


def _positions(es):
    return pl.pallas_call(
        _pos_body,
        in_specs=[pl.BlockSpec((2, T), lambda: (0, 0))],
        out_specs=[
            pl.BlockSpec((2, T), lambda: (0, 0)),
            pl.BlockSpec((1, NT), lambda: (0, 0)),
        ],
        out_shape=[
            jax.ShapeDtypeStruct((2, T), jnp.int32),
            jax.ShapeDtypeStruct((1, NT), jnp.int32),
        ],
    )(es)


def _routing(x1, norm2_w, norm2_b, gate_w):
    t_ln, es, gwt = _gate(x1, norm2_w, norm2_b, gate_w)
    es = es.transpose(1, 0, 2).reshape(2, T)
    gwt = gwt.transpose(1, 0, 2).reshape(2, T)
    pos, tile_map = _positions(es)
    return t_ln, pos, gwt, tile_map


_SC_MESH = None


def _sc_mesh():
    global _SC_MESH
    if _SC_MESH is None:
        _SC_MESH = plsc.VectorSubcoreMesh(core_axis_name="core",
                                          subcore_axis_name="subcore")
    return _SC_MESH


def _sc_scatter(t_ln, pos):
    SW = 128

    @functools.partial(
        pl.kernel,
        out_type=jax.ShapeDtypeStruct((L, DP), jnp.float32),
        mesh=_sc_mesh())
    def scatter_kernel(t_hbm, p_hbm, o_hbm):
        def body(x_vmem, i_vmem):
            pltpu.sync_copy(x_vmem, o_hbm.at[i_vmem.at[0]])

        pltpu.emit_pipeline(
            body,
            grid=(2, T // SW),
            in_specs=[
                pl.BlockSpec((SW, DP), lambda k, i: (i, 0)),
                pl.BlockSpec((1, SW), lambda k, i: (k, i)),
            ],
            out_specs=[],
            core_axis_name=("core", "subcore"),
            dimension_semantics=(pltpu.PARALLEL, pltpu.PARALLEL),
        )(t_hbm, p_hbm)

    return scatter_kernel(t_ln, pos)


def _sc_gather(y, pos_flat):
    GW = 128

    @functools.partial(
        pl.kernel,
        out_type=jax.ShapeDtypeStruct((2 * T, DP), jnp.float32),
        mesh=_sc_mesh())
    def gather_kernel(y_hbm, p_hbm, o_hbm):
        def body(i_vmem, o_vmem):
            pltpu.sync_copy(y_hbm.at[i_vmem.at[0]], o_vmem)

        pltpu.emit_pipeline(
            body,
            grid=(2 * T // GW,),
            in_specs=[pl.BlockSpec((1, GW), lambda i: (0, i))],
            out_specs=[pl.BlockSpec((GW, DP), lambda i: (i, 0))],
            core_axis_name=("core", "subcore"),
            dimension_semantics=(pltpu.PARALLEL,),
        )(p_hbm, o_hbm)

    return gather_kernel(y, pos_flat)


def _gmm_body(tm_ref, xg_ref, w1t_ref, b1_ref, w2t_ref, b2_ref, o_ref):
    x = xg_ref[...]
    h = _gelu(jnp.dot(x, w1t_ref[0], preferred_element_type=jnp.float32)
              + b1_ref[0, 0])
    y = jnp.dot(h, w2t_ref[0], preferred_element_type=jnp.float32) + b2_ref[0, 0]
    o_ref[...] = jnp.concatenate(
        [y, jnp.zeros((BT, DP - DIM), jnp.float32)], axis=-1)


def _grouped_mlp(tile_map, xg, w1, b1, w2, b2):
    w1t = jnp.transpose(w1, (0, 2, 1))
    w1t = jnp.pad(w1t, ((0, 0), (0, DP - DIM), (0, 0)))
    w2t = jnp.transpose(w2, (0, 2, 1))
    grid_spec = pltpu.PrefetchScalarGridSpec(
        num_scalar_prefetch=1,
        grid=(NT,),
        in_specs=[
            pl.BlockSpec((BT, DP), lambda i, tm: (i, 0)),
            pl.BlockSpec((1, DP, DH), lambda i, tm: (tm[i], 0, 0)),
            pl.BlockSpec((1, 1, DH), lambda i, tm: (tm[i], 0, 0)),
            pl.BlockSpec((1, DH, DIM), lambda i, tm: (tm[i], 0, 0)),
            pl.BlockSpec((1, 1, DIM), lambda i, tm: (tm[i], 0, 0)),
        ],
        out_specs=pl.BlockSpec((BT, DP), lambda i, tm: (i, 0)),
    )
    return pl.pallas_call(
        _gmm_body,
        grid_spec=grid_spec,
        out_shape=jax.ShapeDtypeStruct((L, DP), jnp.float32),
        compiler_params=pltpu.CompilerParams(
            dimension_semantics=("arbitrary",)),
    )(tile_map.reshape(NT), xg, w1t, b1.reshape(E, 1, DH), w2t,
      b2.reshape(E, 1, DIM))


def _moe_sparse(x1, norm2_w, norm2_b, gate_w, w1, b1, w2, b2):
    t_ln, pos, gwt, tile_map = _routing(x1, norm2_w, norm2_b, gate_w)
    xg = _sc_scatter(t_ln, pos)
    y = _grouped_mlp(tile_map, xg, w1, b1, w2, b2)
    yg = _sc_gather(y, pos.reshape(1, 2 * T))
    return (x1 + gwt[0][:, None] * yg[:T, :DIM]
            + gwt[1][:, None] * yg[T:, :DIM])


def kernel(x, mask_matrix, norm1_w, norm1_b, qkv_w, qkv_b, rpb, proj_w,
           proj_b, norm2_w, norm2_b, gate_w, w1, b1, w2, b2):
    del mask_matrix
    B, D, H, W, C = x.shape
    wd, wh, ww = WS
    xw = x.reshape(B, D // wd, wd, H // wh, wh, W // ww, ww, C)
    xw = xw.transpose(0, 1, 3, 5, 2, 4, 6, 7).reshape(NW, N, C)
    bias = jnp.take(rpb, _REL_FLAT, axis=0).reshape(N, N, NH).transpose(2, 0, 1)
    x1w = _attention(xw, bias, norm1_w, norm1_b, qkv_w, qkv_b, proj_w, proj_b)
    x1 = x1w.reshape(B, D // wd, H // wh, W // ww, wd, wh, ww, C)
    x1 = x1.transpose(0, 1, 4, 2, 5, 3, 6, 7).reshape(B, D, H, W, C)
    out = _moe_sparse(x1.reshape(T, C), norm2_w, norm2_b, gate_w, w1, b1, w2, b2)
    return out.reshape(B, D, H, W, C)
